# baseline (speedup 1.0000x reference)
import functools

import jax
import jax.numpy as jnp
from jax import lax
from jax.experimental import pallas as pl
from jax.experimental.pallas import tpu as pltpu
from jax.experimental.pallas import tpu_sc as plsc

N = 10000
IN = 128
OUT = 16
EC = 16
H = 128

NC = 2
NS = 16
NW = NC * NS
C = 128
NCH = 20
EP = NW * NCH * C
NSL = 4
SCH = NCH // NSL
ESL = EP // NSL
NP = 10240
SLAB = NP // NS

MSG_W = 128
BE = 2560


@functools.cache
def _gather_sc(sl):
    mesh = plsc.VectorSubcoreMesh(core_axis_name="c", subcore_axis_name="s")

    @functools.partial(
        pl.kernel,
        out_type=jax.ShapeDtypeStruct((SCH, NW, C, IN), jnp.float32),
        mesh=mesh,
        scratch_types=[
            pltpu.VMEM((SCH, 1, C), jnp.int32),
            *[pltpu.VMEM((C, IN), jnp.float32) for _ in range(SCH)],
            *[pltpu.SemaphoreType.DMA for _ in range(2 * SCH)],
        ],
    )
    def gather(x_hbm, row_hbm, xr_hbm, idx_all, *rest):
        bufs = rest[:SCH]
        sg = rest[SCH:2 * SCH]
        ss = rest[2 * SCH:3 * SCH]
        c = lax.axis_index("c")
        s = lax.axis_index("s")
        w = s * NC + c
        pltpu.sync_copy(row_hbm.at[pl.ds(sl * SCH, SCH), w], idx_all)
        g = [
            pltpu.async_copy(x_hbm.at[idx_all.at[j, 0]], bufs[j], sg[j])
            for j in range(SCH)
        ]
        st = [None] * SCH
        for j in range(SCH):
            g[j].wait()
            st[j] = pltpu.async_copy(bufs[j], xr_hbm.at[j, w], ss[j])
        for j in range(SCH):
            st[j].wait()

    return gather


NB_S = 2


@functools.cache
def _scatter_sc():
    mesh = plsc.VectorSubcoreMesh(core_axis_name="c", subcore_axis_name="s")

    @functools.partial(
        pl.kernel,
        out_type=jax.ShapeDtypeStruct((NC, NP, MSG_W), jnp.float32),
        mesh=mesh,
        scratch_types=[
            pltpu.VMEM((NCH, 1, C), jnp.int32),
            *[pltpu.VMEM((C, MSG_W), jnp.float32) for _ in range(NB_S)],
            pltpu.VMEM_SHARED((NP, MSG_W), jnp.float32),
            *[pltpu.SemaphoreType.DMA for _ in range(2 * NB_S)],
        ],
    )
    def scatter(m0, m1, m2, m3, col_hbm, zero_hbm, out_hbm, idx_all, *rest):
        msgs = (m0, m1, m2, m3)
        bufs = rest[:NB_S]
        acc_shared = rest[NB_S]
        sm = rest[NB_S + 1:NB_S + 1 + NB_S]
        sa = rest[NB_S + 1 + NB_S:]
        c = lax.axis_index("c")
        s = lax.axis_index("s")
        w = s * NC + c
        slab = pl.ds(s * SLAB, SLAB)

        def msg_ref(j):
            return msgs[j // SCH].at[j % SCH, w]

        m = [None] * NB_S
        a = [None] * NB_S
        for j in range(NB_S):
            m[j] = pltpu.async_copy(msg_ref(j), bufs[j], sm[j])
        pltpu.sync_copy(col_hbm.at[pl.ds(0, NCH), w], idx_all)
        pltpu.sync_copy(zero_hbm, acc_shared.at[slab])
        plsc.subcore_barrier()
        for j in range(NCH):
            b = j % NB_S
            m[b].wait()
            a[b] = pltpu.async_copy(bufs[b], acc_shared.at[idx_all.at[j, 0]],
                                    sa[b], add=True)
            jn = j + NB_S
            if jn < NCH:
                a[b].wait()
                m[b] = pltpu.async_copy(msg_ref(jn), bufs[b], sm[b])
        for j in range(NCH - NB_S, NCH):
            a[j % NB_S].wait()
        plsc.subcore_barrier()
        pltpu.sync_copy(acc_shared.at[slab], out_hbm.at[c, slab])

    return scatter


def _messages_body(ea_ref, xr_ref, w1_ref, b1_ref, w2p_ref, b2r_ref, out_ref):
    h = jnp.maximum(jnp.dot(ea_ref[...], w1_ref[...]) + b1_ref[...], 0.0)
    kp = jnp.dot(h.astype(jnp.bfloat16), w2p_ref[...],
                 preferred_element_type=jnp.float32)
    xr = xr_ref[...]
    parts = [
        jnp.sum(xr * kp[:, o * IN:(o + 1) * IN], axis=1, keepdims=True)
        for o in range(OUT)
    ]
    m = jnp.concatenate(parts, axis=1) + jnp.dot(xr, b2r_ref[...])
    pad = jnp.concatenate(
        [jnp.ones((BE, 1), jnp.float32), jnp.zeros((BE, MSG_W - OUT - 1), jnp.float32)],
        axis=1,
    )
    out_ref[...] = jnp.concatenate([m, pad], axis=1)


def _messages_tc(ea_full, xr_s, W1, b1_2d, W2p, B2r, sl):
    off = sl * (ESL // BE)
    return pl.pallas_call(
        _messages_body,
        grid=(ESL // BE,),
        in_specs=[
            pl.BlockSpec((BE, EC), lambda i: (i + off, 0)),
            pl.BlockSpec((BE, IN), lambda i: (i, 0)),
            pl.BlockSpec((EC, H), lambda i: (0, 0)),
            pl.BlockSpec((1, H), lambda i: (0, 0)),
            pl.BlockSpec((H, OUT * IN), lambda i: (0, 0)),
            pl.BlockSpec((IN, OUT), lambda i: (0, 0)),
        ],
        out_specs=pl.BlockSpec((BE, MSG_W), lambda i: (i, 0)),
        out_shape=jax.ShapeDtypeStruct((ESL, MSG_W), jnp.float32),
    )(ea_full, xr_s, W1, b1_2d, W2p, B2r)


def _finalize_body(acc_ref, x_ref, wr_ref, br_ref, out_ref):
    a = acc_ref[0] + acc_ref[1]
    m = a[:, :OUT]
    deg = a[:, OUT:OUT + 1]
    root = jnp.dot(x_ref[...], wr_ref[...]) + br_ref[...]
    out_ref[...] = m / jnp.maximum(deg, 1.0) + root


def _finalize_tc(acc, x, Wr, br_2d, bn=1000):
    return pl.pallas_call(
        _finalize_body,
        grid=(N // bn,),
        in_specs=[
            pl.BlockSpec((NC, bn, MSG_W), lambda i: (0, i, 0)),
            pl.BlockSpec((bn, IN), lambda i: (i, 0)),
            pl.BlockSpec((IN, OUT), lambda i: (0, 0)),
            pl.BlockSpec((1, OUT), lambda i: (0, 0)),
        ],
        out_specs=pl.BlockSpec((bn, OUT), lambda i: (i, 0)),
        out_shape=jax.ShapeDtypeStruct((N, OUT), jnp.float32),
    )(acc, x, Wr, br_2d)


def kernel(x, edge_index, edge_attr, W1, b1, W2, b2, Wr, br):
    e = edge_index.shape[1]
    npad = EP - e
    row = jnp.concatenate([edge_index[0], jnp.zeros((npad,), jnp.int32)])
    col = jnp.concatenate([edge_index[1], jnp.full((npad,), N, jnp.int32)])
    ea = jnp.concatenate([edge_attr, jnp.zeros((npad, EC), jnp.float32)], axis=0)
    row4 = row.reshape(NCH, NW, 1, C)
    col4 = col.reshape(NCH, NW, 1, C)
    W2p = (W2.reshape(H, IN, OUT).transpose(0, 2, 1)
           .reshape(H, OUT * IN).astype(jnp.bfloat16))
    B2r = b2.reshape(IN, OUT)
    b1_2d = b1.reshape(1, H)
    br_2d = br.reshape(1, OUT)

    msgs = []
    for sl in range(NSL):
        xr_s = _gather_sc(sl)(x, row4).reshape(ESL, IN)
        m_s = _messages_tc(ea, xr_s, W1, b1_2d, W2p, B2r, sl)
        msgs.append(m_s.reshape(SCH, NW, C, MSG_W))

    zero = jnp.zeros((SLAB, MSG_W), jnp.float32)
    acc = _scatter_sc()(*msgs, col4, zero)
    return _finalize_tc(acc, x, Wr, br_2d)


# speedup vs baseline: 1.2022x; 1.2022x over previous
import functools

import jax
import jax.numpy as jnp
from jax import lax
from jax.experimental import pallas as pl
from jax.experimental.pallas import tpu as pltpu
from jax.experimental.pallas import tpu_sc as plsc

N = 10000
IN = 128
OUT = 16
EC = 16
H = 128

NC = 2
NS = 16
NW = NC * NS
C = 128
NCH = 20
EP = NW * NCH * C
NSL = 4
SCH = NCH // NSL
ESL = EP // NSL
NP = 10240
SLAB = NP // NS

MSG_W = 128
BE = 2560


@functools.cache
def _gather_sc():
    mesh = plsc.VectorSubcoreMesh(core_axis_name="c", subcore_axis_name="s")

    @functools.partial(
        pl.kernel,
        out_type=jax.ShapeDtypeStruct((NW, SCH, C, IN), jnp.float32),
        mesh=mesh,
        scratch_types=[
            pltpu.VMEM((SCH, C), jnp.int32),
            *[pltpu.VMEM((C, IN), jnp.float32) for _ in range(SCH)],
            *[pltpu.SemaphoreType.DMA for _ in range(2 * SCH)],
        ],
    )
    def gather(x_hbm, row_hbm, xr_hbm, idx_all, *rest):
        bufs = rest[:SCH]
        sg = rest[SCH:2 * SCH]
        ss = rest[2 * SCH:3 * SCH]
        c = lax.axis_index("c")
        s = lax.axis_index("s")
        w = s * NC + c
        pltpu.sync_copy(row_hbm.at[w], idx_all)
        g = [
            pltpu.async_copy(x_hbm.at[idx_all.at[j]], bufs[j], sg[j])
            for j in range(SCH)
        ]
        st = [None] * SCH
        for j in range(SCH):
            g[j].wait()
            st[j] = pltpu.async_copy(bufs[j], xr_hbm.at[w, j], ss[j])
        for j in range(SCH):
            st[j].wait()

    return gather


NB_S = 2


@functools.cache
def _scatter_sc():
    mesh = plsc.VectorSubcoreMesh(core_axis_name="c", subcore_axis_name="s")

    @functools.partial(
        pl.kernel,
        out_type=jax.ShapeDtypeStruct((NC, NP, MSG_W), jnp.float32),
        mesh=mesh,
        scratch_types=[
            pltpu.VMEM((NCH, C), jnp.int32),
            *[pltpu.VMEM((C, MSG_W), jnp.float32) for _ in range(NB_S)],
            pltpu.VMEM_SHARED((NP, MSG_W), jnp.float32),
            *[pltpu.SemaphoreType.DMA for _ in range(2 * NB_S)],
        ],
    )
    def scatter(m0, m1, m2, m3, col_hbm, zero_hbm, out_hbm, idx_all, *rest):
        msgs = (m0, m1, m2, m3)
        bufs = rest[:NB_S]
        acc_shared = rest[NB_S]
        sm = rest[NB_S + 1:NB_S + 1 + NB_S]
        sa = rest[NB_S + 1 + NB_S:]
        c = lax.axis_index("c")
        s = lax.axis_index("s")
        w = s * NC + c
        slab = pl.ds(s * SLAB, SLAB)

        def msg_ref(j):
            return msgs[j // SCH].at[w, j % SCH]

        m = [None] * NB_S
        a = [None] * NB_S
        for j in range(NB_S):
            m[j] = pltpu.async_copy(msg_ref(j), bufs[j], sm[j])
        pltpu.sync_copy(col_hbm.at[w], idx_all)
        pltpu.sync_copy(zero_hbm.at[slab], acc_shared.at[slab])
        plsc.subcore_barrier()
        for j in range(NCH):
            b = j % NB_S
            m[b].wait()
            a[b] = pltpu.async_copy(bufs[b], acc_shared.at[idx_all.at[j]], sa[b],
                                    add=True)
            jn = j + NB_S
            if jn < NCH:
                a[b].wait()
                m[b] = pltpu.async_copy(msg_ref(jn), bufs[b], sm[b])
        for j in range(NCH - NB_S, NCH):
            a[j % NB_S].wait()
        plsc.subcore_barrier()
        pltpu.sync_copy(acc_shared.at[slab], out_hbm.at[c, slab])

    return scatter


def _messages_body(ea_ref, xr_ref, w1_ref, b1_ref, w2p_ref, b2r_ref, out_ref):
    ea = ea_ref[...].T
    h = jnp.maximum(jnp.dot(ea, w1_ref[...]) + b1_ref[...], 0.0)
    kp = jnp.dot(h.astype(jnp.bfloat16), w2p_ref[...],
                 preferred_element_type=jnp.float32)
    xr = xr_ref[...]
    parts = [
        jnp.sum(xr * kp[:, o * IN:(o + 1) * IN], axis=1, keepdims=True)
        for o in range(OUT)
    ]
    m = jnp.concatenate(parts, axis=1) + jnp.dot(xr, b2r_ref[...])
    pad = jnp.concatenate(
        [jnp.ones((BE, 1), jnp.float32), jnp.zeros((BE, MSG_W - OUT - 1), jnp.float32)],
        axis=1,
    )
    out_ref[...] = jnp.concatenate([m, pad], axis=1)


def _messages_tc(ea_ts, xr_s, W1, b1_2d, W2p, B2r):
    return pl.pallas_call(
        _messages_body,
        grid=(ESL // BE,),
        in_specs=[
            pl.BlockSpec((EC, BE), lambda i: (0, i)),
            pl.BlockSpec((BE, IN), lambda i: (i, 0)),
            pl.BlockSpec((EC, H), lambda i: (0, 0)),
            pl.BlockSpec((1, H), lambda i: (0, 0)),
            pl.BlockSpec((H, OUT * IN), lambda i: (0, 0)),
            pl.BlockSpec((IN, OUT), lambda i: (0, 0)),
        ],
        out_specs=pl.BlockSpec((BE, MSG_W), lambda i: (i, 0)),
        out_shape=jax.ShapeDtypeStruct((ESL, MSG_W), jnp.float32),
    )(ea_ts, xr_s, W1, b1_2d, W2p, B2r)


def _finalize_body(acc_ref, x_ref, wr_ref, br_ref, out_ref):
    a = acc_ref[0] + acc_ref[1]
    m = a[:, :OUT]
    deg = a[:, OUT:OUT + 1]
    root = jnp.dot(x_ref[...], wr_ref[...]) + br_ref[...]
    out_ref[...] = m / jnp.maximum(deg, 1.0) + root


def _finalize_tc(acc, x, Wr, br_2d, bn=1000):
    return pl.pallas_call(
        _finalize_body,
        grid=(N // bn,),
        in_specs=[
            pl.BlockSpec((NC, bn, MSG_W), lambda i: (0, i, 0)),
            pl.BlockSpec((bn, IN), lambda i: (i, 0)),
            pl.BlockSpec((IN, OUT), lambda i: (0, 0)),
            pl.BlockSpec((1, OUT), lambda i: (0, 0)),
        ],
        out_specs=pl.BlockSpec((bn, OUT), lambda i: (i, 0)),
        out_shape=jax.ShapeDtypeStruct((N, OUT), jnp.float32),
    )(acc, x, Wr, br_2d)


def kernel(x, edge_index, edge_attr, W1, b1, W2, b2, Wr, br):
    e = edge_index.shape[1]
    npad = EP - e
    row = jnp.concatenate([edge_index[0], jnp.zeros((npad,), jnp.int32)])
    col = jnp.concatenate([edge_index[1], jnp.full((npad,), N, jnp.int32)])
    ea_t4 = jnp.pad(edge_attr.T, ((0, 0), (0, npad))).reshape(EC, NW, NCH, C)
    row3 = row.reshape(NW, NCH, C)
    col3 = col.reshape(NW, NCH, C)
    W2p = (W2.reshape(H, IN, OUT).transpose(0, 2, 1)
           .reshape(H, OUT * IN).astype(jnp.bfloat16))
    B2r = b2.reshape(IN, OUT)
    b1_2d = b1.reshape(1, H)
    br_2d = br.reshape(1, OUT)

    gather = _gather_sc()
    msgs = []
    for sl in range(NSL):
        row_s = lax.slice_in_dim(row3, sl * SCH, (sl + 1) * SCH, axis=1)
        xr_s = gather(x, row_s).reshape(ESL, IN)
        ea_ts = lax.slice_in_dim(ea_t4, sl * SCH, (sl + 1) * SCH,
                                 axis=2).reshape(EC, ESL)
        m_s = _messages_tc(ea_ts, xr_s, W1, b1_2d, W2p, B2r)
        msgs.append(m_s.reshape(NW, SCH, C, MSG_W))

    zero = jnp.zeros((NP, MSG_W), jnp.float32)
    acc = _scatter_sc()(*msgs, col3, zero)
    return _finalize_tc(acc, x, Wr, br_2d)


# speedup vs baseline: 1.2034x; 1.0010x over previous
import functools

import jax
import jax.numpy as jnp
from jax import lax
from jax.experimental import pallas as pl
from jax.experimental.pallas import tpu as pltpu
from jax.experimental.pallas import tpu_sc as plsc

N = 10000
IN = 128
OUT = 16
EC = 16
H = 128

NC = 2
NS = 16
NW = NC * NS
C = 128
NCH = 20
EP = NW * NCH * C
NSL = 4
SCH = NCH // NSL
ESL = EP // NSL
NP = 10240
SLAB = NP // NS

MSG_W = 128
ACC_W = 32
BE = 2560


@functools.cache
def _gather_sc():
    mesh = plsc.VectorSubcoreMesh(core_axis_name="c", subcore_axis_name="s")

    @functools.partial(
        pl.kernel,
        out_type=jax.ShapeDtypeStruct((NW, SCH, C, IN), jnp.float32),
        mesh=mesh,
        scratch_types=[
            pltpu.VMEM((SCH, C), jnp.int32),
            *[pltpu.VMEM((C, IN), jnp.float32) for _ in range(SCH)],
            *[pltpu.SemaphoreType.DMA for _ in range(2 * SCH)],
        ],
    )
    def gather(x_hbm, row_hbm, xr_hbm, idx_all, *rest):
        bufs = rest[:SCH]
        sg = rest[SCH:2 * SCH]
        ss = rest[2 * SCH:3 * SCH]
        c = lax.axis_index("c")
        s = lax.axis_index("s")
        w = s * NC + c
        pltpu.sync_copy(row_hbm.at[w], idx_all)
        g = [
            pltpu.async_copy(x_hbm.at[idx_all.at[j]], bufs[j], sg[j])
            for j in range(SCH)
        ]
        st = [None] * SCH
        for j in range(SCH):
            g[j].wait()
            st[j] = pltpu.async_copy(bufs[j], xr_hbm.at[w, j], ss[j])
        for j in range(SCH):
            st[j].wait()

    return gather


NB_S = 2


@functools.cache
def _scatter_sc():
    mesh = plsc.VectorSubcoreMesh(core_axis_name="c", subcore_axis_name="s")

    @functools.partial(
        pl.kernel,
        out_type=jax.ShapeDtypeStruct((NC, NP, MSG_W), jnp.float32),
        mesh=mesh,
        scratch_types=[
            pltpu.VMEM((NCH, C), jnp.int32),
            *[pltpu.VMEM((C, MSG_W), jnp.float32) for _ in range(NB_S)],
            pltpu.VMEM_SHARED((NP, MSG_W), jnp.float32),
            *[pltpu.SemaphoreType.DMA for _ in range(2 * NB_S)],
        ],
    )
    def scatter(m0, m1, m2, m3, col_hbm, zero_hbm, out_hbm, idx_all, *rest):
        msgs = (m0, m1, m2, m3)
        bufs = rest[:NB_S]
        acc_shared = rest[NB_S]
        sm = rest[NB_S + 1:NB_S + 1 + NB_S]
        sa = rest[NB_S + 1 + NB_S:]
        c = lax.axis_index("c")
        s = lax.axis_index("s")
        w = s * NC + c
        slab = pl.ds(s * SLAB, SLAB)

        def msg_ref(j):
            return msgs[j // SCH].at[w, j % SCH]

        m = [None] * NB_S
        a = [None] * NB_S
        for j in range(NB_S):
            m[j] = pltpu.async_copy(msg_ref(j), bufs[j], sm[j])
        pltpu.sync_copy(col_hbm.at[w], idx_all)
        pltpu.sync_copy(zero_hbm, acc_shared.at[slab])
        plsc.subcore_barrier()
        for j in range(NCH):
            b = j % NB_S
            m[b].wait()
            a[b] = pltpu.async_copy(bufs[b], acc_shared.at[idx_all.at[j]], sa[b],
                                    add=True)
            jn = j + NB_S
            if jn < NCH:
                a[b].wait()
                m[b] = pltpu.async_copy(msg_ref(jn), bufs[b], sm[b])
        for j in range(NCH - NB_S, NCH):
            a[j % NB_S].wait()
        plsc.subcore_barrier()
        pltpu.sync_copy(acc_shared.at[slab], out_hbm.at[c, slab])

    return scatter


def _messages_body(ea_ref, xr_ref, w1_ref, b1_ref, w2p_ref, b2r_ref, out_ref):
    ea = ea_ref[...].T
    h = jnp.maximum(jnp.dot(ea, w1_ref[...]) + b1_ref[...], 0.0)
    kp = jnp.dot(h.astype(jnp.bfloat16), w2p_ref[...],
                 preferred_element_type=jnp.float32)
    xr = xr_ref[...]
    parts = [
        jnp.sum(xr * kp[:, o * IN:(o + 1) * IN], axis=1, keepdims=True)
        for o in range(OUT)
    ]
    m = jnp.concatenate(parts, axis=1) + jnp.dot(xr, b2r_ref[...])
    pad = jnp.concatenate(
        [jnp.ones((BE, 1), jnp.float32), jnp.zeros((BE, MSG_W - OUT - 1), jnp.float32)],
        axis=1,
    )
    out_ref[...] = jnp.concatenate([m, pad], axis=1)


def _messages_tc(ea_ts, xr_s, W1, b1_2d, W2p, B2r):
    return pl.pallas_call(
        _messages_body,
        grid=(ESL // BE,),
        in_specs=[
            pl.BlockSpec((EC, BE), lambda i: (0, i)),
            pl.BlockSpec((BE, IN), lambda i: (i, 0)),
            pl.BlockSpec((EC, H), lambda i: (0, 0)),
            pl.BlockSpec((1, H), lambda i: (0, 0)),
            pl.BlockSpec((H, OUT * IN), lambda i: (0, 0)),
            pl.BlockSpec((IN, OUT), lambda i: (0, 0)),
        ],
        out_specs=pl.BlockSpec((BE, MSG_W), lambda i: (i, 0)),
        out_shape=jax.ShapeDtypeStruct((ESL, MSG_W), jnp.float32),
    )(ea_ts, xr_s, W1, b1_2d, W2p, B2r)


def _finalize_body(acc_ref, x_ref, wr_ref, br_ref, out_ref):
    a = acc_ref[0] + acc_ref[1]
    m = a[:, :OUT]
    deg = a[:, OUT:OUT + 1]
    root = jnp.dot(x_ref[...], wr_ref[...]) + br_ref[...]
    out_ref[...] = m / jnp.maximum(deg, 1.0) + root


def _finalize_tc(acc, x, Wr, br_2d, bn=1000):
    return pl.pallas_call(
        _finalize_body,
        grid=(N // bn,),
        in_specs=[
            pl.BlockSpec((NC, bn, MSG_W), lambda i: (0, i, 0)),
            pl.BlockSpec((bn, IN), lambda i: (i, 0)),
            pl.BlockSpec((IN, OUT), lambda i: (0, 0)),
            pl.BlockSpec((1, OUT), lambda i: (0, 0)),
        ],
        out_specs=pl.BlockSpec((bn, OUT), lambda i: (i, 0)),
        out_shape=jax.ShapeDtypeStruct((N, OUT), jnp.float32),
    )(acc, x, Wr, br_2d)


def kernel(x, edge_index, edge_attr, W1, b1, W2, b2, Wr, br):
    e = edge_index.shape[1]
    npad = EP - e
    row = jnp.concatenate([edge_index[0], jnp.zeros((npad,), jnp.int32)])
    col = jnp.concatenate([edge_index[1], jnp.full((npad,), N, jnp.int32)])
    ea_t4 = jnp.pad(edge_attr.T, ((0, 0), (0, npad))).reshape(EC, NW, NCH, C)
    row3 = row.reshape(NW, NCH, C)
    col3 = col.reshape(NW, NCH, C)
    W2p = (W2.reshape(H, IN, OUT).transpose(0, 2, 1)
           .reshape(H, OUT * IN).astype(jnp.bfloat16))
    B2r = b2.reshape(IN, OUT)
    b1_2d = b1.reshape(1, H)
    br_2d = br.reshape(1, OUT)

    gather = _gather_sc()
    msgs = []
    for sl in range(NSL):
        row_s = lax.slice_in_dim(row3, sl * SCH, (sl + 1) * SCH, axis=1)
        xr_s = gather(x, row_s).reshape(ESL, IN)
        ea_ts = lax.slice_in_dim(ea_t4, sl * SCH, (sl + 1) * SCH,
                                 axis=2).reshape(EC, ESL)
        m_s = _messages_tc(ea_ts, xr_s, W1, b1_2d, W2p, B2r)
        msgs.append(m_s.reshape(NW, SCH, C, MSG_W))

    zero = jnp.zeros((SLAB, MSG_W), jnp.float32)
    acc = _scatter_sc()(*msgs, col3, zero)
    return _finalize_tc(acc, x, Wr, br_2d)


# speedup vs baseline: 1.2057x; 1.0019x over previous
import functools

import jax
import jax.numpy as jnp
from jax import lax
from jax.experimental import pallas as pl
from jax.experimental.pallas import tpu as pltpu
from jax.experimental.pallas import tpu_sc as plsc

N = 10000
IN = 128
OUT = 16
EC = 16
H = 128

NC = 2
NS = 16
NW = NC * NS
C = 128
NCH = 20
EP = NW * NCH * C
NSL = 4
SCH = NCH // NSL
ESL = EP // NSL
NP = 10240
SLAB = NP // NS

MSG_W = 128
BE = 2560


@functools.cache
def _gather_sc():
    mesh = plsc.VectorSubcoreMesh(core_axis_name="c", subcore_axis_name="s")

    @functools.partial(
        pl.kernel,
        out_type=jax.ShapeDtypeStruct((NW, SCH, C, IN), jnp.float32),
        mesh=mesh,
        scratch_types=[
            pltpu.VMEM((SCH, C), jnp.int32),
            *[pltpu.VMEM((C, IN), jnp.float32) for _ in range(SCH)],
            *[pltpu.SemaphoreType.DMA for _ in range(2 * SCH)],
        ],
    )
    def gather(x_hbm, row_hbm, xr_hbm, idx_all, *rest):
        bufs = rest[:SCH]
        sg = rest[SCH:2 * SCH]
        ss = rest[2 * SCH:3 * SCH]
        c = lax.axis_index("c")
        s = lax.axis_index("s")
        w = s * NC + c
        pltpu.sync_copy(row_hbm.at[w], idx_all)
        g = [
            pltpu.async_copy(x_hbm.at[idx_all.at[j]], bufs[j], sg[j])
            for j in range(SCH)
        ]
        st = [None] * SCH
        for j in range(SCH):
            g[j].wait()
            st[j] = pltpu.async_copy(bufs[j], xr_hbm.at[w, j], ss[j])
        for j in range(SCH):
            st[j].wait()

    return gather


NB_S = 2


@functools.cache
def _scatter_sc():
    mesh = plsc.VectorSubcoreMesh(core_axis_name="c", subcore_axis_name="s")

    @functools.partial(
        pl.kernel,
        out_type=jax.ShapeDtypeStruct((NC, NP, MSG_W), jnp.float32),
        mesh=mesh,
        scratch_types=[
            pltpu.VMEM((NCH, C), jnp.int32),
            *[pltpu.VMEM((C, MSG_W), jnp.float32) for _ in range(NB_S)],
            pltpu.VMEM_SHARED((NP, MSG_W), jnp.float32),
            *[pltpu.SemaphoreType.DMA for _ in range(2 * NB_S)],
        ],
    )
    def scatter(m0, m1, m2, m3, col_hbm, zero_hbm, out_hbm, idx_all, *rest):
        msgs = (m0, m1, m2, m3)
        bufs = rest[:NB_S]
        acc_shared = rest[NB_S]
        sm = rest[NB_S + 1:NB_S + 1 + NB_S]
        sa = rest[NB_S + 1 + NB_S:]
        c = lax.axis_index("c")
        s = lax.axis_index("s")
        w = s * NC + c
        slab = pl.ds(s * SLAB, SLAB)

        def msg_ref(j):
            return msgs[j // SCH].at[w, j % SCH]

        m = [None] * NB_S
        a = [None] * NB_S
        for j in range(NB_S):
            m[j] = pltpu.async_copy(msg_ref(j), bufs[j], sm[j])
        pltpu.sync_copy(col_hbm.at[w], idx_all)
        pltpu.sync_copy(zero_hbm, acc_shared.at[slab])
        plsc.subcore_barrier()
        for j in range(NCH):
            b = j % NB_S
            m[b].wait()
            a[b] = pltpu.async_copy(bufs[b], acc_shared.at[idx_all.at[j]], sa[b],
                                    add=True)
            jn = j + NB_S
            if jn < NCH:
                a[b].wait()
                m[b] = pltpu.async_copy(msg_ref(jn), bufs[b], sm[b])
        for j in range(NCH - NB_S, NCH):
            a[j % NB_S].wait()
        plsc.subcore_barrier()
        pltpu.sync_copy(acc_shared.at[slab], out_hbm.at[c, slab])

    return scatter


def _messages_body(ea_ref, xr_ref, w1_ref, b1_ref, w2p_ref, b2r_ref, out_ref):
    ea = ea_ref[...].T
    h = jnp.maximum(jnp.dot(ea, w1_ref[...]) + b1_ref[...], 0.0)
    kp = jnp.dot(h.astype(jnp.bfloat16), w2p_ref[...],
                 preferred_element_type=jnp.float32)
    xr = xr_ref[...]
    parts = [
        jnp.sum(xr * kp[:, o * IN:(o + 1) * IN], axis=1, keepdims=True)
        for o in range(OUT)
    ]
    m = jnp.concatenate(parts, axis=1) + jnp.dot(xr, b2r_ref[...])
    pad = jnp.concatenate(
        [jnp.ones((BE, 1), jnp.float32), jnp.zeros((BE, MSG_W - OUT - 1), jnp.float32)],
        axis=1,
    )
    out_ref[...] = jnp.concatenate([m, pad], axis=1)


def _messages_tc(ea_ts, xr_s, W1, b1_2d, W2p, B2r):
    return pl.pallas_call(
        _messages_body,
        grid=(ESL // BE,),
        in_specs=[
            pl.BlockSpec((EC, BE), lambda i: (0, i)),
            pl.BlockSpec((BE, IN), lambda i: (i, 0)),
            pl.BlockSpec((EC, H), lambda i: (0, 0)),
            pl.BlockSpec((1, H), lambda i: (0, 0)),
            pl.BlockSpec((H, OUT * IN), lambda i: (0, 0)),
            pl.BlockSpec((IN, OUT), lambda i: (0, 0)),
        ],
        out_specs=pl.BlockSpec((BE, MSG_W), lambda i: (i, 0)),
        out_shape=jax.ShapeDtypeStruct((ESL, MSG_W), jnp.float32),
    )(ea_ts, xr_s, W1, b1_2d, W2p, B2r)


def _finalize_body(acc_ref, x_ref, wr_ref, br_ref, out_ref):
    a = acc_ref[0] + acc_ref[1]
    m = a[:, :OUT]
    deg = a[:, OUT:OUT + 1]
    root = jnp.dot(x_ref[...], wr_ref[...]) + br_ref[...]
    out_ref[...] = m / jnp.maximum(deg, 1.0) + root


def _finalize_tc(acc, x, Wr, br_2d, bn=1000):
    return pl.pallas_call(
        _finalize_body,
        grid=(N // bn,),
        in_specs=[
            pl.BlockSpec((NC, bn, MSG_W), lambda i: (0, i, 0)),
            pl.BlockSpec((bn, IN), lambda i: (i, 0)),
            pl.BlockSpec((IN, OUT), lambda i: (0, 0)),
            pl.BlockSpec((1, OUT), lambda i: (0, 0)),
        ],
        out_specs=pl.BlockSpec((bn, OUT), lambda i: (i, 0)),
        out_shape=jax.ShapeDtypeStruct((N, OUT), jnp.float32),
    )(acc, x, Wr, br_2d)


def kernel(x, edge_index, edge_attr, W1, b1, W2, b2, Wr, br):
    e = edge_index.shape[1]
    npad = EP - e
    row = jnp.concatenate([edge_index[0], jnp.zeros((npad,), jnp.int32)])
    col = jnp.concatenate([edge_index[1], jnp.full((npad,), N, jnp.int32)])
    ea_t4 = jnp.pad(edge_attr.T, ((0, 0), (0, npad))).reshape(EC, NW, NCH, C)
    row3 = row.reshape(NW, NCH, C)
    col3 = col.reshape(NW, NCH, C)
    W2p = (W2.reshape(H, IN, OUT).transpose(0, 2, 1)
           .reshape(H, OUT * IN).astype(jnp.bfloat16))
    B2r = b2.reshape(IN, OUT)
    b1_2d = b1.reshape(1, H)
    br_2d = br.reshape(1, OUT)

    gather = _gather_sc()
    msgs = []
    for sl in range(NSL):
        row_s = lax.slice_in_dim(row3, sl * SCH, (sl + 1) * SCH, axis=1)
        xr_s = gather(x, row_s).reshape(ESL, IN)
        ea_ts = lax.slice_in_dim(ea_t4, sl * SCH, (sl + 1) * SCH,
                                 axis=2).reshape(EC, ESL)
        m_s = _messages_tc(ea_ts, xr_s, W1, b1_2d, W2p, B2r)
        msgs.append(m_s.reshape(NW, SCH, C, MSG_W))

    zero = jnp.zeros((SLAB, MSG_W), jnp.float32)
    acc = _scatter_sc()(*msgs, col3, zero)
    return _finalize_tc(acc, x, Wr, br_2d)
